# baseline (device time: 13954 ns/iter reference)
import jax
import jax.numpy as jnp
from jax import lax
from jax.experimental import pallas as pl
from jax.experimental.pallas import tpu as pltpu

N_DEV = 32

PHASES = ((1,), (7, 4, 3), (24, 16, 8))
ALL_MASKS = tuple(m for ph in PHASES for m in ph)
N_FLOWS = len(ALL_MASKS)


def kernel(x):
    m_per, n = x.shape

    def body(x_ref, out_ref, acc_ref, comm_ref, send_sems, recv_sems):
        my = lax.axis_index("i")

        barrier_sem = pltpu.get_barrier_semaphore()
        for mask in ALL_MASKS:
            pl.semaphore_signal(
                barrier_sem, inc=1,
                device_id=(my ^ mask,), device_id_type=pl.DeviceIdType.MESH,
            )

        acc_ref[:, :] = jnp.max(x_ref[:, :], axis=0, keepdims=True)

        pl.semaphore_wait(barrier_sem, N_FLOWS)

        slot = 0
        for ph, masks in enumerate(PHASES):
            rdmas = []
            for mask in masks:
                rdma = pltpu.make_async_remote_copy(
                    src_ref=acc_ref,
                    dst_ref=comm_ref.at[slot],
                    send_sem=send_sems.at[slot],
                    recv_sem=recv_sems.at[slot],
                    device_id=(my ^ mask,),
                    device_id_type=pl.DeviceIdType.MESH,
                )
                rdma.start()
                rdmas.append(rdma)
                slot += 1
            for rdma in rdmas:
                rdma.wait_recv()
            for rdma in rdmas:
                rdma.wait_send()
            lo, hi = slot - len(masks), slot
            phase_max = jnp.max(comm_ref[lo:hi, 0, :], axis=0, keepdims=True)
            if ph == len(PHASES) - 1:
                out_ref[:, :] = jnp.maximum(acc_ref[:, :], phase_max)
            else:
                acc_ref[:, :] = jnp.maximum(acc_ref[:, :], phase_max)

    return pl.pallas_call(
        body,
        out_shape=jax.ShapeDtypeStruct((1, n), jnp.float32),
        in_specs=[pl.BlockSpec(memory_space=pltpu.VMEM)],
        out_specs=pl.BlockSpec(memory_space=pltpu.VMEM),
        scratch_shapes=[
            pltpu.VMEM((1, n), jnp.float32),
            pltpu.VMEM((N_FLOWS, 1, n), jnp.float32),
            pltpu.SemaphoreType.DMA((N_FLOWS,)),
            pltpu.SemaphoreType.DMA((N_FLOWS,)),
        ],
        compiler_params=pltpu.CompilerParams(collective_id=0),
    )(x)


# device time: 12929 ns/iter; 1.0793x vs baseline; 1.0793x over previous
import jax
import jax.numpy as jnp
from jax import lax
from jax.experimental import pallas as pl
from jax.experimental.pallas import tpu as pltpu

N_DEV = 32

PHASES = ((7, 6, 5, 4, 3, 2, 1), (24, 16, 8))
ALL_MASKS = tuple(m for ph in PHASES for m in ph)
N_FLOWS = len(ALL_MASKS)


def kernel(x):
    m_per, n = x.shape

    def body(x_ref, out_ref, acc_ref, comm_ref, send_sems, recv_sems):
        my = lax.axis_index("i")

        barrier_sem = pltpu.get_barrier_semaphore()
        for mask in ALL_MASKS:
            pl.semaphore_signal(
                barrier_sem, inc=1,
                device_id=(my ^ mask,), device_id_type=pl.DeviceIdType.MESH,
            )

        acc_ref[:, :] = jnp.max(x_ref[:, :], axis=0, keepdims=True)

        pl.semaphore_wait(barrier_sem, N_FLOWS)

        slot = 0
        for ph, masks in enumerate(PHASES):
            rdmas = []
            for mask in masks:
                rdma = pltpu.make_async_remote_copy(
                    src_ref=acc_ref,
                    dst_ref=comm_ref.at[slot],
                    send_sem=send_sems.at[slot],
                    recv_sem=recv_sems.at[slot],
                    device_id=(my ^ mask,),
                    device_id_type=pl.DeviceIdType.MESH,
                )
                rdma.start()
                rdmas.append(rdma)
                slot += 1
            for rdma in rdmas:
                rdma.wait_recv()
            for rdma in rdmas:
                rdma.wait_send()
            lo, hi = slot - len(masks), slot
            phase_max = jnp.max(comm_ref[lo:hi, 0, :], axis=0, keepdims=True)
            if ph == len(PHASES) - 1:
                out_ref[:, :] = jnp.maximum(acc_ref[:, :], phase_max)
            else:
                acc_ref[:, :] = jnp.maximum(acc_ref[:, :], phase_max)

    return pl.pallas_call(
        body,
        out_shape=jax.ShapeDtypeStruct((1, n), jnp.float32),
        in_specs=[pl.BlockSpec(memory_space=pltpu.VMEM)],
        out_specs=pl.BlockSpec(memory_space=pltpu.VMEM),
        scratch_shapes=[
            pltpu.VMEM((1, n), jnp.float32),
            pltpu.VMEM((N_FLOWS, 1, n), jnp.float32),
            pltpu.SemaphoreType.DMA((N_FLOWS,)),
            pltpu.SemaphoreType.DMA((N_FLOWS,)),
        ],
        compiler_params=pltpu.CompilerParams(collective_id=0),
    )(x)


# device time: 12443 ns/iter; 1.1214x vs baseline; 1.0391x over previous
import jax
import jax.numpy as jnp
from jax import lax
from jax.experimental import pallas as pl
from jax.experimental.pallas import tpu as pltpu

N_DEV = 32


def kernel(x):
    m_per, n = x.shape

    def body(x_ref, out_ref, acc_ref, comm_ref, send_sems, recv_sems):
        my = lax.axis_index("i")

        barrier_sem = pltpu.get_barrier_semaphore()
        for j in range(N_DEV - 1):
            peer = (my + j + 1) % N_DEV
            pl.semaphore_signal(
                barrier_sem, inc=1,
                device_id=(peer,), device_id_type=pl.DeviceIdType.MESH,
            )

        acc_ref[:, :] = jnp.max(x_ref[:, :], axis=0, keepdims=True)

        pl.semaphore_wait(barrier_sem, N_DEV - 1)

        rdmas = []
        for j in range(N_DEV - 1):
            peer = (my + j + 1) % N_DEV
            rdma = pltpu.make_async_remote_copy(
                src_ref=acc_ref,
                dst_ref=comm_ref.at[j],
                send_sem=send_sems.at[j],
                recv_sem=recv_sems.at[j],
                device_id=(peer,),
                device_id_type=pl.DeviceIdType.MESH,
            )
            rdma.start()
            rdmas.append(rdma)

        for rdma in rdmas:
            rdma.wait_recv()

        out_ref[:, :] = jnp.maximum(
            acc_ref[:, :],
            jnp.max(comm_ref[:, 0, :], axis=0, keepdims=True),
        )

        for rdma in rdmas:
            rdma.wait_send()

    return pl.pallas_call(
        body,
        out_shape=jax.ShapeDtypeStruct((1, n), jnp.float32),
        in_specs=[pl.BlockSpec(memory_space=pltpu.VMEM)],
        out_specs=pl.BlockSpec(memory_space=pltpu.VMEM),
        scratch_shapes=[
            pltpu.VMEM((1, n), jnp.float32),
            pltpu.VMEM((N_DEV - 1, 1, n), jnp.float32),
            pltpu.SemaphoreType.DMA((N_DEV - 1,)),
            pltpu.SemaphoreType.DMA((N_DEV - 1,)),
        ],
        compiler_params=pltpu.CompilerParams(collective_id=0),
    )(x)
